# Initial kernel scaffold; baseline (speedup 1.0000x reference)
import jax
import jax.numpy as jnp
from jax.experimental import pallas as pl


def kernel(x, dw_w, dw_b, sp_w, sp_b, w1, b1, w2, b2):
    raise NotImplementedError("write your pallas kernel here")


# speedup vs baseline: 1.0238x; 1.0238x over previous
import functools

import jax
import jax.numpy as jnp
from jax.experimental import pallas as pl
from jax.experimental.pallas import tpu as pltpu


def _ecbam_body(x_ref, wm_ref, dwb_ref, spk_ref, spb_ref,
                w1_ref, b1_ref, w2_ref, b2_ref, out_ref,
                *, PB, C, H, W):
    HW = H * W
    x = x_ref[0]

    def sh(v, s):
        return pltpu.roll(v, shift=(-s) % HW, axis=2)

    rm = sh(x, -1)
    rp = sh(x, +1)
    a_m1 = wm_ref[0] * rm + wm_ref[1] * x + wm_ref[2] * rp
    a_0 = wm_ref[3] * rm + wm_ref[4] * x + wm_ref[5] * rp
    a_p1 = wm_ref[6] * rm + wm_ref[7] * x + wm_ref[8] * rp
    xc = a_0 + sh(a_m1, -W) + sh(a_p1, +W) + dwb_ref[...]

    maps_avg = jnp.mean(xc, axis=1)
    maps_max = jnp.max(xc, axis=1)
    sm = jnp.concatenate([maps_avg, maps_max], axis=1)

    sconv = jnp.dot(sm, spk_ref[...],
                    precision=jax.lax.Precision.HIGHEST,
                    preferred_element_type=jnp.float32)
    sa = jax.nn.sigmoid(sconv + spb_ref[0])

    y = xc * sa[:, None, :]

    avg_c = jnp.mean(y, axis=2, keepdims=True)
    max_c = jnp.max(y, axis=2, keepdims=True)
    z1 = w1_ref[0] * avg_c + w1_ref[1] * max_c
    h = jnp.maximum(jnp.sum(z1, axis=1) + b1_ref[...], 0.0)
    ca = jax.nn.sigmoid(jnp.sum(w2_ref[...] * h[:, None, :], axis=2,
                                keepdims=True) + b2_ref[...])

    out_ref[0] = (y * ca).astype(out_ref.dtype)


def _pix(H, W):
    yy = jnp.repeat(jnp.arange(H, dtype=jnp.int32), W)
    xx = jnp.tile(jnp.arange(W, dtype=jnp.int32), H)
    return yy, xx


def _tap_masks(H, W):
    yy, xx = _pix(H, W)
    masks = []
    for dy in (-1, 0, 1):
        for dx in (-1, 0, 1):
            m = jnp.ones((H * W,), jnp.bool_)
            if dx == -1:
                m = m & (xx >= 1)
            elif dx == 1:
                m = m & (xx <= W - 2)
            if dy == -1:
                m = m & (yy <= H - 2)
            elif dy == 1:
                m = m & (yy >= 1)
            masks.append(m.astype(jnp.float32))
    return jnp.stack(masks, axis=0)


def _conv_matrix(w2d, H, W):
    K = w2d.shape[0]
    r = K // 2
    yy, xx = _pix(H, W)
    dY = yy[:, None] - yy[None, :]
    dX = xx[:, None] - xx[None, :]
    inb = (jnp.abs(dY) <= r) & (jnp.abs(dX) <= r)
    iy = jnp.clip(dY + r, 0, K - 1)
    ix = jnp.clip(dX + r, 0, K - 1)
    return jnp.where(inb, w2d[iy, ix], 0.0).astype(jnp.float32)


def kernel(x, dw_w, dw_b, sp_w, sp_b, w1, b1, w2, b2):
    B, C, H, W = x.shape
    hidden = w1.shape[0]
    HW = H * W
    PB = next(p for p in (8, 4, 2, 1) if B % p == 0)

    x4 = x.astype(jnp.float32).reshape(B // PB, PB, C, HW)

    w3 = dw_w.reshape(C, 9).astype(jnp.float32)
    wm = jnp.transpose(w3)[:, :, None] * _tap_masks(H, W)[:, None, :]
    dwb = dw_b.reshape(C, 1).astype(jnp.float32)

    spk = jnp.concatenate(
        [_conv_matrix(sp_w[0, 0].astype(jnp.float32), H, W),
         _conv_matrix(sp_w[0, 1].astype(jnp.float32), H, W)], axis=0)
    spb = sp_b.reshape(1).astype(jnp.float32)

    w1s = jnp.stack([jnp.transpose(w1[:, :C]),
                     jnp.transpose(w1[:, C:])], axis=0).astype(jnp.float32)
    b1r = b1.reshape(1, hidden).astype(jnp.float32)
    w2c = w2.astype(jnp.float32)
    b2c = b2.reshape(C, 1).astype(jnp.float32)

    body = functools.partial(_ecbam_body, PB=PB, C=C, H=H, W=W)

    out4 = pl.pallas_call(
        body,
        out_shape=jax.ShapeDtypeStruct((B // PB, PB, C, HW), jnp.float32),
        grid=(B // PB,),
        in_specs=[
            pl.BlockSpec((1, PB, C, HW), lambda b: (b, 0, 0, 0)),
            pl.BlockSpec((9, C, HW), lambda b: (0, 0, 0)),
            pl.BlockSpec((C, 1), lambda b: (0, 0)),
            pl.BlockSpec((2 * HW, HW), lambda b: (0, 0)),
            pl.BlockSpec(memory_space=pltpu.MemorySpace.SMEM),
            pl.BlockSpec((2, C, hidden), lambda b: (0, 0, 0)),
            pl.BlockSpec((1, hidden), lambda b: (0, 0)),
            pl.BlockSpec((C, hidden), lambda b: (0, 0)),
            pl.BlockSpec((C, 1), lambda b: (0, 0)),
        ],
        out_specs=pl.BlockSpec((1, PB, C, HW), lambda b: (b, 0, 0, 0)),
        compiler_params=pltpu.CompilerParams(
            dimension_semantics=("parallel",)),
    )(x4, wm, dwb, spk, spb, w1s, b1r, w2c, b2c)

    return out4.reshape(B, C, H, W)


# speedup vs baseline: 49.3141x; 48.1689x over previous
import functools

import jax
import jax.numpy as jnp
from jax.experimental import pallas as pl
from jax.experimental.pallas import tpu as pltpu


def _ecbam_body(x_ref, wm_ref, dwb_ref, sp_ref, spb_ref,
                w1_ref, b1_ref, w2_ref, b2_ref, out_ref,
                *, PB, C, H, W):
    HW = H * W
    x = x_ref[0]

    def sh(v, s):
        return pltpu.roll(v, shift=(-s) % HW, axis=v.ndim - 1)

    rm = sh(x, -1)
    rp = sh(x, +1)
    a_m1 = wm_ref[0] * rm + wm_ref[1] * x + wm_ref[2] * rp
    a_0 = wm_ref[3] * rm + wm_ref[4] * x + wm_ref[5] * rp
    a_p1 = wm_ref[6] * rm + wm_ref[7] * x + wm_ref[8] * rp
    xc = a_0 + sh(a_m1, -W) + sh(a_p1, +W) + dwb_ref[...]

    maps = jnp.stack([jnp.mean(xc, axis=1),
                      jnp.max(xc, axis=1)], axis=0)

    rd = [sh(maps, dx) for dx in range(-3, 0)] + [maps] + \
         [sh(maps, dx) for dx in range(1, 4)]
    sacc = None
    for dy in range(-3, 4):
        ady = None
        for dx in range(-3, 4):
            k = (dy + 3) * 7 + (dx + 3)
            t = sp_ref[k][:, None, :] * rd[dx + 3]
            ady = t if ady is None else ady + t
        term = ady if dy == 0 else sh(ady, dy * W)
        sacc = term if sacc is None else sacc + term
    sa = jax.nn.sigmoid(sacc[0] + sacc[1] + spb_ref[0])

    y = xc * sa[:, None, :]

    avg_c = jnp.mean(y, axis=2, keepdims=True)
    max_c = jnp.max(y, axis=2, keepdims=True)
    z1 = w1_ref[0] * avg_c + w1_ref[1] * max_c
    h = jnp.maximum(jnp.sum(z1, axis=1) + b1_ref[...], 0.0)
    ca = jax.nn.sigmoid(jnp.sum(w2_ref[...] * h[:, None, :], axis=2,
                                keepdims=True) + b2_ref[...])

    out_ref[0] = (y * ca).astype(out_ref.dtype)


def _folded_masks(H, W, r):
    yy = jnp.repeat(jnp.arange(H, dtype=jnp.int32), W)
    xx = jnp.tile(jnp.arange(W, dtype=jnp.int32), H)
    masks = []
    for dy in range(-r, r + 1):
        for dx in range(-r, r + 1):
            m = jnp.ones((H * W,), jnp.bool_)
            if dx < 0:
                m = m & (xx >= -dx)
            elif dx > 0:
                m = m & (xx <= W - 1 - dx)
            if dy < 0:
                m = m & (yy <= H - 1 + dy)
            elif dy > 0:
                m = m & (yy >= dy)
            masks.append(m.astype(jnp.float32))
    return jnp.stack(masks, axis=0)


def kernel(x, dw_w, dw_b, sp_w, sp_b, w1, b1, w2, b2):
    B, C, H, W = x.shape
    hidden = w1.shape[0]
    HW = H * W
    PB = next(p for p in (8, 4, 2, 1) if B % p == 0)

    x4 = x.astype(jnp.float32).reshape(B // PB, PB, C, HW)

    w3 = dw_w.reshape(C, 9).astype(jnp.float32)
    wm = jnp.transpose(w3)[:, :, None] * _folded_masks(H, W, 1)[:, None, :]
    dwb = dw_b.reshape(C, 1).astype(jnp.float32)

    w7 = sp_w.reshape(2, 49).astype(jnp.float32)
    sp = jnp.transpose(w7)[:, :, None] * _folded_masks(H, W, 3)[:, None, :]
    spb = sp_b.reshape(1).astype(jnp.float32)

    w1s = jnp.stack([jnp.transpose(w1[:, :C]),
                     jnp.transpose(w1[:, C:])], axis=0).astype(jnp.float32)
    b1r = b1.reshape(1, hidden).astype(jnp.float32)
    w2c = w2.astype(jnp.float32)
    b2c = b2.reshape(C, 1).astype(jnp.float32)

    body = functools.partial(_ecbam_body, PB=PB, C=C, H=H, W=W)

    out4 = pl.pallas_call(
        body,
        out_shape=jax.ShapeDtypeStruct((B // PB, PB, C, HW), jnp.float32),
        grid=(B // PB,),
        in_specs=[
            pl.BlockSpec((1, PB, C, HW), lambda b: (b, 0, 0, 0)),
            pl.BlockSpec((9, C, HW), lambda b: (0, 0, 0)),
            pl.BlockSpec((C, 1), lambda b: (0, 0)),
            pl.BlockSpec((49, 2, HW), lambda b: (0, 0, 0)),
            pl.BlockSpec(memory_space=pltpu.MemorySpace.SMEM),
            pl.BlockSpec((2, C, hidden), lambda b: (0, 0, 0)),
            pl.BlockSpec((1, hidden), lambda b: (0, 0)),
            pl.BlockSpec((C, hidden), lambda b: (0, 0)),
            pl.BlockSpec((C, 1), lambda b: (0, 0)),
        ],
        out_specs=pl.BlockSpec((1, PB, C, HW), lambda b: (b, 0, 0, 0)),
        compiler_params=pltpu.CompilerParams(
            dimension_semantics=("parallel",)),
    )(x4, wm, dwb, sp, spb, w1s, b1r, w2c, b2c)

    return out4.reshape(B, C, H, W)


# speedup vs baseline: 49.9453x; 1.0128x over previous
import functools

import jax
import jax.numpy as jnp
from jax import lax
from jax.experimental import pallas as pl
from jax.experimental.pallas import tpu as pltpu


def _ecbam_body(x_ref, wm_ref, dwb_ref, sp_ref, spb_ref,
                w1_ref, b1_ref, w2_ref, b2_ref, out_ref,
                *, PB, C, H, W):
    G = H // 8
    L = G * W
    xw = x_ref[0]
    x = jnp.concatenate([xw[:, :, 8 * j:8 * j + 8, :] for j in range(G)],
                        axis=3)

    si = lax.broadcasted_iota(jnp.int32, (1, 1, 8, 1), 2)

    def lane_sh(v, dx):
        return pltpu.roll(v, shift=(-dx) % L, axis=3)

    def row_sh(v, d):
        if d > 0:
            a = pltpu.roll(v, shift=8 - d, axis=2)
            b = pltpu.roll(a, shift=L - W, axis=3)
            return jnp.where(si <= 7 - d, a, b)
        m = -d
        a = pltpu.roll(v, shift=m, axis=2)
        b = pltpu.roll(a, shift=W, axis=3)
        return jnp.where(si >= m, a, b)

    rm = lane_sh(x, -1)
    rp = lane_sh(x, +1)
    a_m1 = wm_ref[0] * rm + wm_ref[1] * x + wm_ref[2] * rp
    xc = row_sh(a_m1, -1) + dwb_ref[...].reshape(1, C, 1, 1)
    a_p1 = wm_ref[6] * rm + wm_ref[7] * x + wm_ref[8] * rp
    xc = xc + row_sh(a_p1, +1)
    xc = xc + (wm_ref[3] * rm + wm_ref[4] * x + wm_ref[5] * rp)

    maps = jnp.stack([jnp.mean(xc, axis=1),
                      jnp.max(xc, axis=1)], axis=0)

    rd = [lane_sh(maps, dx) for dx in range(-3, 0)] + [maps] + \
         [lane_sh(maps, dx) for dx in range(1, 4)]
    sacc = None
    for dy in range(-3, 4):
        ady = None
        for dx in range(-3, 4):
            k = (dy + 3) * 7 + (dx + 3)
            t = sp_ref[k][:, None] * rd[dx + 3]
            ady = t if ady is None else ady + t
        term = ady if dy == 0 else row_sh(ady, dy)
        sacc = term if sacc is None else sacc + term
    sa = jax.nn.sigmoid(sacc[0] + sacc[1] + spb_ref[0])

    y = xc * sa[:, None]

    avg_c = jnp.mean(y, axis=(2, 3))
    max_c = jnp.max(y, axis=(2, 3))
    am = jnp.concatenate([avg_c, max_c], axis=1)
    hp = lax.dot_general(am, w1_ref[...], (((1,), (1,)), ((), ())),
                         precision=jax.lax.Precision.HIGHEST,
                         preferred_element_type=jnp.float32)
    h = jnp.maximum(hp + b1_ref[...], 0.0)
    cp = lax.dot_general(h, w2_ref[...], (((1,), (1,)), ((), ())),
                         precision=jax.lax.Precision.HIGHEST,
                         preferred_element_type=jnp.float32)
    ca = jax.nn.sigmoid(cp + b2_ref[...])

    out = (y * ca[:, :, None, None]).astype(out_ref.dtype)
    for j in range(G):
        out_ref[0, :, :, 8 * j:8 * j + 8, :] = out[:, :, :, W * j:W * j + W]


def _folded_masks(H, W, r):
    yy = jnp.repeat(jnp.arange(H, dtype=jnp.int32), W)
    xx = jnp.tile(jnp.arange(W, dtype=jnp.int32), H)
    masks = []
    for dy in range(-r, r + 1):
        for dx in range(-r, r + 1):
            m = jnp.ones((H * W,), jnp.bool_)
            if dx < 0:
                m = m & (xx >= -dx)
            elif dx > 0:
                m = m & (xx <= W - 1 - dx)
            if dy < 0:
                m = m & (yy <= H - 1 + dy)
            elif dy > 0:
                m = m & (yy >= dy)
            masks.append(m.astype(jnp.float32))
    return jnp.stack(masks, axis=0)


def _to_packed(planes, H, W):
    G = H // 8
    lead = planes.shape[:-1]
    p = planes.reshape(lead + (G, 8, W))
    p = jnp.moveaxis(p, -3, -2)
    return p.reshape(lead + (8, G * W))


def kernel(x, dw_w, dw_b, sp_w, sp_b, w1, b1, w2, b2):
    B, C, H, W = x.shape
    hidden = w1.shape[0]
    PB = next(p for p in (8, 4, 2, 1) if B % p == 0)
    G = H // 8
    L = G * W

    x5 = x.astype(jnp.float32).reshape(B // PB, PB, C, H, W)

    w3 = dw_w.reshape(C, 9).astype(jnp.float32)
    m3 = _to_packed(_folded_masks(H, W, 1), H, W)
    wm = jnp.transpose(w3)[:, :, None, None] * m3[:, None]
    dwb = dw_b.reshape(C, 1).astype(jnp.float32)

    w7 = sp_w.reshape(2, 49).astype(jnp.float32)
    m7 = _to_packed(_folded_masks(H, W, 3), H, W)
    sp = jnp.transpose(w7)[:, :, None, None] * m7[:, None]
    spb = sp_b.reshape(1).astype(jnp.float32)

    w1r = w1.astype(jnp.float32)
    b1r = b1.reshape(1, hidden).astype(jnp.float32)
    w2c = w2.astype(jnp.float32)
    b2r = b2.reshape(1, C).astype(jnp.float32)

    body = functools.partial(_ecbam_body, PB=PB, C=C, H=H, W=W)

    out5 = pl.pallas_call(
        body,
        out_shape=jax.ShapeDtypeStruct((B // PB, PB, C, H, W), jnp.float32),
        grid=(B // PB,),
        in_specs=[
            pl.BlockSpec((1, PB, C, H, W), lambda b: (b, 0, 0, 0, 0)),
            pl.BlockSpec((9, C, 8, L), lambda b: (0, 0, 0, 0)),
            pl.BlockSpec((C, 1), lambda b: (0, 0)),
            pl.BlockSpec((49, 2, 8, L), lambda b: (0, 0, 0, 0)),
            pl.BlockSpec(memory_space=pltpu.MemorySpace.SMEM),
            pl.BlockSpec((hidden, 2 * C), lambda b: (0, 0)),
            pl.BlockSpec((1, hidden), lambda b: (0, 0)),
            pl.BlockSpec((C, hidden), lambda b: (0, 0)),
            pl.BlockSpec((1, C), lambda b: (0, 0)),
        ],
        out_specs=pl.BlockSpec((1, PB, C, H, W), lambda b: (b, 0, 0, 0, 0)),
        compiler_params=pltpu.CompilerParams(
            dimension_semantics=("parallel",)),
    )(x5, wm, dwb, sp, spb, w1r, b1r, w2c, b2r)

    return out5.reshape(B, C, H, W)


# speedup vs baseline: 117.3712x; 2.3500x over previous
import functools

import jax
import jax.numpy as jnp
from jax import lax
from jax.experimental import pallas as pl
from jax.experimental.pallas import tpu as pltpu


def _body(x_ref, w3_ref, dwb_ref, w7_ref, spb_ref,
          w1_ref, b1_ref, w2_ref, b2_ref, out_ref,
          xc_ref, maps_ref, sa_ref, stats_ref, ca_ref,
          *, C, CB, H, W, BB):
    ph = pl.program_id(1)
    c = pl.program_id(2)
    NC = C // CB
    HW = H * W

    wi = lax.broadcasted_iota(jnp.int32, (1, 1, W, 1), 2)
    hi = lax.broadcasted_iota(jnp.int32, (1, H, 1, 1), 1)

    def shw(v, dx):
        r = pltpu.roll(v, shift=(-dx) % W, axis=2)
        if dx < 0:
            return jnp.where(wi >= -dx, r, 0.0)
        return jnp.where(wi <= W - 1 - dx, r, 0.0)

    def shh(v, dy):
        r = pltpu.roll(v, shift=(-dy) % H, axis=1)
        if dy < 0:
            return jnp.where(hi >= -dy, r, 0.0)
        return jnp.where(hi <= H - 1 - dy, r, 0.0)

    @pl.when(ph == 0)
    def _phase0():
        x = x_ref[...]
        rm = shw(x, -1)
        rp = shw(x, +1)

        def w3c(k):
            return w3_ref[k].reshape(CB, 1, 1, 1)

        xc = None
        for dy in (-1, 0, 1):
            a = (w3c((dy + 1) * 3 + 0) * rm + w3c((dy + 1) * 3 + 1) * x
                 + w3c((dy + 1) * 3 + 2) * rp)
            a = a if dy == 0 else shh(a, dy)
            xc = a if xc is None else xc + a
        xc = xc + dwb_ref[...].reshape(CB, 1, 1, 1)
        xc_ref[pl.ds(c * CB, CB)] = xc.astype(xc_ref.dtype)

        ps = jnp.sum(xc, axis=0, keepdims=True)
        pm = jnp.max(xc, axis=0, keepdims=True)

        @pl.when(c == 0)
        def _():
            maps_ref[0:1] = ps
            maps_ref[1:2] = pm

        @pl.when(c > 0)
        def _():
            maps_ref[0:1] += ps
            maps_ref[1:2] = jnp.maximum(maps_ref[1:2], pm)

    @pl.when(jnp.logical_and(ph == 1, c == 0))
    def _spatial():
        sconv = None
        for comp in range(2):
            m = maps_ref[comp:comp + 1]
            if comp == 0:
                m = m * (1.0 / C)
            rd = [shw(m, dx) for dx in range(-3, 0)] + [m] + \
                 [shw(m, dx) for dx in range(1, 4)]
            for dy in range(-3, 4):
                ady = None
                for dx in range(-3, 4):
                    k = (dy + 3) * 7 + (dx + 3)
                    t = w7_ref[k, comp].reshape(1, 1, 1, 1) * rd[dx + 3]
                    ady = t if ady is None else ady + t
                term = ady if dy == 0 else shh(ady, dy)
                sconv = term if sconv is None else sconv + term
        sa_ref[...] = jax.nn.sigmoid(sconv[0] + spb_ref[0])

    @pl.when(ph == 1)
    def _phase1():
        y = xc_ref[pl.ds(c * CB, CB)].astype(jnp.float32) * sa_ref[...][None]
        stats_ref[pl.ds(c * CB, CB)] = jnp.sum(y, axis=(1, 2)) * (1.0 / HW)
        stats_ref[pl.ds(C + c * CB, CB)] = jnp.max(y, axis=(1, 2))

    @pl.when(jnp.logical_and(ph == 2, c == 0))
    def _channel():
        hp = lax.dot_general(w1_ref[...], stats_ref[...],
                             (((1,), (0,)), ((), ())),
                             precision=jax.lax.Precision.HIGHEST,
                             preferred_element_type=jnp.float32)
        h = jnp.maximum(hp + b1_ref[...], 0.0)
        cp = lax.dot_general(w2_ref[...], h, (((1,), (0,)), ((), ())),
                             precision=jax.lax.Precision.HIGHEST,
                             preferred_element_type=jnp.float32)
        ca_ref[...] = jax.nn.sigmoid(cp + b2_ref[...])

    @pl.when(ph == 2)
    def _phase2():
        y = xc_ref[pl.ds(c * CB, CB)].astype(jnp.float32) * sa_ref[...][None]
        ca = ca_ref[pl.ds(c * CB, CB)]
        out_ref[...] = (y * ca[:, None, None, :]).astype(out_ref.dtype)


def kernel(x, dw_w, dw_b, sp_w, sp_b, w1, b1, w2, b2):
    B, C, H, W = x.shape
    hidden = w1.shape[0]
    BB = next(b for b in (128, 64, 32, 16, 8, 4, 2, 1) if B % b == 0)
    CB = next(cc for cc in (8, 4, 2, 1) if C % cc == 0)
    NC = C // CB

    xt = jnp.transpose(x.astype(jnp.float32), (1, 2, 3, 0))

    w3 = dw_w.reshape(C, 9).astype(jnp.float32)
    w9 = jnp.transpose(w3).reshape(9, C, 1)
    dwb = dw_b.reshape(C, 1).astype(jnp.float32)
    w7 = sp_w.reshape(2, 49).astype(jnp.float32)
    w49 = jnp.transpose(w7).reshape(49, 2, 1)
    spb = sp_b.reshape(1).astype(jnp.float32)
    w1r = w1.astype(jnp.float32)
    b1r = b1.reshape(hidden, 1).astype(jnp.float32)
    w2c = w2.astype(jnp.float32)
    b2r = b2.reshape(C, 1).astype(jnp.float32)

    body = functools.partial(_body, C=C, CB=CB, H=H, W=W, BB=BB)

    out_t = pl.pallas_call(
        body,
        out_shape=jax.ShapeDtypeStruct((C, H, W, B), jnp.float32),
        grid=(B // BB, 3, NC),
        in_specs=[
            pl.BlockSpec((CB, H, W, BB),
                         lambda b, ph, c: (jnp.where(ph == 0, c, NC - 1),
                                           0, 0, b)),
            pl.BlockSpec((9, CB, 1), lambda b, ph, c: (0, c, 0)),
            pl.BlockSpec((CB, 1), lambda b, ph, c: (c, 0)),
            pl.BlockSpec((49, 2, 1), lambda b, ph, c: (0, 0, 0)),
            pl.BlockSpec(memory_space=pltpu.MemorySpace.SMEM),
            pl.BlockSpec((hidden, 2 * C), lambda b, ph, c: (0, 0)),
            pl.BlockSpec((hidden, 1), lambda b, ph, c: (0, 0)),
            pl.BlockSpec((C, hidden), lambda b, ph, c: (0, 0)),
            pl.BlockSpec((C, 1), lambda b, ph, c: (0, 0)),
        ],
        out_specs=pl.BlockSpec((CB, H, W, BB),
                               lambda b, ph, c: (jnp.where(ph == 2, c, 0),
                                                 0, 0, b)),
        scratch_shapes=[
            pltpu.VMEM((C, H, W, BB), jnp.bfloat16),
            pltpu.VMEM((2, H, W, BB), jnp.float32),
            pltpu.VMEM((H, W, BB), jnp.float32),
            pltpu.VMEM((2 * C, BB), jnp.float32),
            pltpu.VMEM((C, BB), jnp.float32),
        ],
        compiler_params=pltpu.CompilerParams(
            dimension_semantics=("arbitrary", "arbitrary", "arbitrary")),
    )(xt, w9, dwb, w49, spb, w1r, b1r, w2c, b2r)

    return jnp.transpose(out_t, (3, 0, 1, 2))
